# baseline (speedup 1.0000x reference)
import functools

import jax
import jax.numpy as jnp
from jax.experimental import pallas as pl
from jax.experimental.pallas import tpu as pltpu

B = 256
LATENT = 50
BUF = 500
BUFP = 512
K = 3
NA = 6
NAP = 8
HID = 512


def _s2d(x_nhwc):
    b, h, w, c = x_nhwc.shape
    xp = jnp.pad(x_nhwc, ((0, 0), (1, 1), (1, 1), (0, 0)))
    hh = (h + 2) // 2
    xr = xp.reshape(b, hh, 2, hh, 2, c)
    return xr.transpose(0, 1, 3, 2, 4, 5).reshape(b, hh, hh, 4 * c)


def _conv_body(xs_ref, w_ref, b_ref, o_ref, *, bt, ho, cin):
    parts = []
    for kh in range(3):
        for kw in range(3):
            off = ((kh % 2) * 2 + (kw % 2)) * cin
            s = xs_ref[:, kh // 2:kh // 2 + ho, kw // 2:kw // 2 + ho,
                       off:off + cin]
            parts.append(s)
    patches = jnp.concatenate(parts, axis=-1).reshape(bt * ho * ho, 9 * cin)
    y = jax.lax.dot_general(patches, w_ref[...], (((1,), (0,)), ((), ())),
                            preferred_element_type=jnp.float32)
    y = y + b_ref[...]
    o_ref[...] = jnp.maximum(y, 0.0).reshape(bt, ho, ho, -1)


def _conv_layer(x_nhwc, w, b, *, bt):
    bsz, h, _, cin = x_nhwc.shape
    cout = w.shape[0]
    ho = h // 2
    hh = ho + 1
    xs = _s2d(x_nhwc)
    wr = w.transpose(2, 3, 1, 0).reshape(9 * cin, cout)
    br = b.reshape(1, cout)
    body = functools.partial(_conv_body, bt=bt, ho=ho, cin=cin)
    return pl.pallas_call(
        body,
        grid=(bsz // bt,),
        in_specs=[pl.BlockSpec((bt, hh, hh, 4 * cin), lambda i: (i, 0, 0, 0)),
                  pl.BlockSpec((9 * cin, cout), lambda i: (0, 0)),
                  pl.BlockSpec((1, cout), lambda i: (0, 0))],
        out_specs=pl.BlockSpec((bt, ho, ho, cout), lambda i: (i, 0, 0, 0)),
        out_shape=jax.ShapeDtypeStruct((bsz, ho, ho, cout), jnp.float32),
    )(xs, wr, br)


def _tail_body(h3_ref, wfc_ref, bfc_ref, g_ref, be_ref, buf_ref, o_ref):
    h = jax.lax.dot_general(h3_ref[...], wfc_ref[...], (((1,), (0,)), ((), ())),
                            preferred_element_type=jnp.float32) + bfc_ref[...]
    mu = jnp.mean(h, axis=1, keepdims=True)
    var = jnp.mean((h - mu) * (h - mu), axis=1, keepdims=True)
    hn = (h - mu) * jax.lax.rsqrt(var + 1e-5) * g_ref[...] + be_ref[...]
    reps = jnp.tanh(hn)
    buft = buf_ref[...]
    rr = jnp.sum(reps * reps, axis=1, keepdims=True)
    bb = jnp.sum(buft * buft, axis=0, keepdims=True)
    d2 = rr + bb - 2.0 * jax.lax.dot_general(
        reps, buft, (((1,), (0,)), ((), ())), preferred_element_type=jnp.float32)
    d = jnp.sqrt(jnp.maximum(d2, 0.0))
    tot = jnp.zeros((B, 1), jnp.float32)
    for _ in range(K):
        m = jnp.min(d, axis=1, keepdims=True)
        tot = tot + m
        i = jnp.argmin(d, axis=1).astype(jnp.int32)
        hit = jax.lax.broadcasted_iota(jnp.int32, (B, BUFP), 1) == i[:, None]
        d = jnp.where(hit, jnp.float32(jnp.inf), d)
    o_ref[...] = -jnp.log(tot * (1.0 / K) + 1e-8)


def _ac_body(flat_ref, wh_ref, bh_ref, wa_ref, ba_ref, wv_ref, bv_ref,
             probs_ref, logp_ref, val_ref, acc_ref, *, nk):
    k = pl.program_id(0)

    @pl.when(k == 0)
    def _():
        acc_ref[...] = jnp.zeros_like(acc_ref)

    acc_ref[...] += jax.lax.dot_general(
        flat_ref[...], wh_ref[...], (((1,), (1,)), ((), ())),
        preferred_element_type=jnp.float32)

    @pl.when(k == nk - 1)
    def _():
        hid = jnp.maximum(acc_ref[...] + bh_ref[...], 0.0)
        logits = jax.lax.dot_general(hid, wa_ref[...], (((1,), (0,)), ((), ())),
                                     preferred_element_type=jnp.float32) + ba_ref[...]
        m = jnp.max(logits, axis=1, keepdims=True)
        e = jnp.exp(logits - m)
        s = jnp.sum(e, axis=1, keepdims=True)
        probs_ref[...] = e / s
        logp_ref[...] = logits - m - jnp.log(s)
        val_ref[...] = jax.lax.dot_general(
            hid, wv_ref[...], (((1,), (0,)), ((), ())),
            preferred_element_type=jnp.float32) + bv_ref[...]


def kernel(x, W1, b1, W2, b2, W3, b3, Wfc, bfc, gamma, beta, buffer,
           Wh, bh, Wa, ba, Wv, bv):
    flat = x.reshape(B, -1)
    kdim = flat.shape[1]
    nk = 8
    kc = kdim // nk
    wa_p = jnp.zeros((HID, NAP), jnp.float32).at[:, :NA].set(Wa.T)
    ba_p = jnp.full((1, NAP), -1e30, jnp.float32).at[:, :NA].set(ba)
    probs_p, logp_p, value = pl.pallas_call(
        functools.partial(_ac_body, nk=nk),
        grid=(nk,),
        in_specs=[
            pl.BlockSpec((B, kc), lambda k: (0, k)),
            pl.BlockSpec((HID, kc), lambda k: (0, k)),
            pl.BlockSpec((1, HID), lambda k: (0, 0)),
            pl.BlockSpec((HID, NAP), lambda k: (0, 0)),
            pl.BlockSpec((1, NAP), lambda k: (0, 0)),
            pl.BlockSpec((HID, 1), lambda k: (0, 0)),
            pl.BlockSpec((1, 1), lambda k: (0, 0)),
        ],
        out_specs=[
            pl.BlockSpec((B, NAP), lambda k: (0, 0)),
            pl.BlockSpec((B, NAP), lambda k: (0, 0)),
            pl.BlockSpec((B, 1), lambda k: (0, 0)),
        ],
        out_shape=[
            jax.ShapeDtypeStruct((B, NAP), jnp.float32),
            jax.ShapeDtypeStruct((B, NAP), jnp.float32),
            jax.ShapeDtypeStruct((B, 1), jnp.float32),
        ],
        scratch_shapes=[pltpu.VMEM((B, HID), jnp.float32)],
    )(flat, Wh, bh.reshape(1, HID), wa_p, ba_p, Wv.T, bv.reshape(1, 1))
    probs = probs_p[:, :NA]
    log_probs = logp_p[:, :NA]

    xt = x.transpose(0, 2, 3, 1)
    h1 = _conv_layer(xt, W1, b1, bt=8)
    h2 = _conv_layer(h1, W2, b2, bt=16)
    h3 = _conv_layer(h2, W3, b3, bt=32)
    h3f = h3.reshape(B, -1)
    wfc_r = Wfc.reshape(LATENT, 128, 10, 10).transpose(2, 3, 1, 0).reshape(-1, LATENT)
    buf_p = jnp.full((BUFP, LATENT), 1e3, jnp.float32).at[:BUF].set(buffer)

    reward = pl.pallas_call(
        _tail_body,
        in_specs=[
            pl.BlockSpec(h3f.shape, lambda: (0, 0)),
            pl.BlockSpec(wfc_r.shape, lambda: (0, 0)),
            pl.BlockSpec((1, LATENT), lambda: (0, 0)),
            pl.BlockSpec((1, LATENT), lambda: (0, 0)),
            pl.BlockSpec((1, LATENT), lambda: (0, 0)),
            pl.BlockSpec((LATENT, BUFP), lambda: (0, 0)),
        ],
        out_specs=pl.BlockSpec((B, 1), lambda: (0, 0)),
        out_shape=jax.ShapeDtypeStruct((B, 1), jnp.float32),
    )(h3f, wfc_r, bfc.reshape(1, LATENT), gamma.reshape(1, LATENT),
      beta.reshape(1, LATENT), buf_p.T)

    return (probs, log_probs, value, reward.reshape(B))


# speedup vs baseline: 3.0893x; 3.0893x over previous
import functools

import jax
import jax.numpy as jnp
from jax.experimental import pallas as pl
from jax.experimental.pallas import tpu as pltpu

B = 256
LATENT = 50
BUF = 500
BUFP = 512
K = 3
NA = 6
NAP = 8
HID = 512
BT = 16
G1 = 21
G1C = 24
G3 = 11
G3C = 16


def _enc_body(xq_ref, w1_ref, w2_ref, w3_ref, b1_ref, b2_ref, b3_ref, o_ref):
    xqm = xq_ref[...].reshape(BT * G1 * G1C, 64)
    d = [[None, None], [None, None]]
    for ia in range(2):
        for ib in range(2):
            d[ia][ib] = jax.lax.dot_general(
                xqm, w1_ref[ia * 2 + ib], (((1,), (0,)), ((), ())),
                preferred_element_type=jnp.float32).reshape(BT, G1, G1C, 128)
    y = d[1][1]
    y = y + jnp.pad(d[0][1][:, :G1 - 1], ((0, 0), (1, 0), (0, 0), (0, 0)))
    y = y + jnp.pad(d[1][0][:, :, :G1C - 1], ((0, 0), (0, 0), (1, 0), (0, 0)))
    y = y + jnp.pad(d[0][0][:, :G1 - 1, :G1C - 1],
                    ((0, 0), (1, 0), (1, 0), (0, 0)))
    y = jnp.maximum(y + b1_ref[...].reshape(1, 1, 1, 128), 0.0)
    ri = jax.lax.broadcasted_iota(jnp.int32, (1, G1, G1C, 128), 1)
    ci = jax.lax.broadcasted_iota(jnp.int32, (1, G1, G1C, 128), 2)
    li = jax.lax.broadcasted_iota(jnp.int32, (1, G1, G1C, 128), 3)
    lp = li // 64
    lq = (li // 32) % 2
    bad = ((lp == 0) & (ri == 0)) | ((lp == 1) & (ri == G1 - 1)) \
        | ((lq == 0) & (ci == 0)) | ((lq == 1) & (ci >= 20))
    xs2 = jnp.where(bad, 0.0, y)

    xs2m = xs2.reshape(BT * G1 * G1C, 128)
    for ia in range(2):
        for ib in range(2):
            d[ia][ib] = jax.lax.dot_general(
                xs2m, w2_ref[ia * 2 + ib], (((1,), (0,)), ((), ())),
                preferred_element_type=jnp.float32).reshape(BT, G1, G1C, 64)
    y2 = (d[0][0][:, :20, :20] + d[0][1][:, :20, 1:21]
          + d[1][0][:, 1:21, :20] + d[1][1][:, 1:21, 1:21])
    y2 = jnp.maximum(y2 + b2_ref[...].reshape(1, 1, 1, 64), 0.0)

    yr = y2.reshape(BT, 10, 2, 20, 64)
    er, orr = yr[:, :, 0], yr[:, :, 1]
    zrow = jnp.zeros((BT, 1, 20, 64), jnp.float32)
    r0 = jnp.concatenate([zrow, orr], axis=1)
    r1 = jnp.concatenate([er, zrow], axis=1)
    xs3p = []
    zc1 = jnp.zeros((BT, G3, 1, 64), jnp.float32)
    zc5 = jnp.zeros((BT, G3, 5, 64), jnp.float32)
    for rsel in (r0, r1):
        rc = rsel.reshape(BT, G3, 10, 2, 64)
        ec, oc = rc[:, :, :, 0], rc[:, :, :, 1]
        xs3p.append(jnp.concatenate([zc1, oc, zc5], axis=2))
        xs3p.append(jnp.concatenate([ec, zc1, zc5], axis=2))
    xs3 = jnp.concatenate(xs3p, axis=-1)

    xs3m = xs3.reshape(BT * G3 * G3C, 256)
    for ia in range(2):
        for ib in range(2):
            d[ia][ib] = jax.lax.dot_general(
                xs3m, w3_ref[ia * 2 + ib], (((1,), (0,)), ((), ())),
                preferred_element_type=jnp.float32).reshape(BT, G3, G3C, 128)
    y3 = (d[0][0][:, :10, :10] + d[0][1][:, :10, 1:11]
          + d[1][0][:, 1:11, :10] + d[1][1][:, 1:11, 1:11])
    y3 = jnp.maximum(y3 + b3_ref[...].reshape(1, 1, 1, 128), 0.0)
    o_ref[...] = y3


def _pack_w1(W1):
    ws = [[jnp.zeros((64, 128), jnp.float32) for _ in range(2)]
          for _ in range(2)]
    for P in range(2):
        for Q in range(2):
            for kh in range(3):
                dr = 2 * P + kh - 2
                ia = 1 if dr >= 0 else 0
                r4 = dr % 4
                for kw in range(3):
                    dc = 2 * Q + kw - 2
                    ib = 1 if dc >= 0 else 0
                    c4 = dc % 4
                    blk = W1[:, :, kh, kw].T
                    row = r4 * 16 + c4 * 4
                    col = (P * 2 + Q) * 32
                    ws[ia][ib] = jax.lax.dynamic_update_slice(
                        ws[ia][ib], blk, (row, col))
    return jnp.stack([ws[0][0], ws[0][1], ws[1][0], ws[1][1]])


def _pack_w23(W, cin, cout):
    ws = [[jnp.zeros((4 * cin, cout), jnp.float32) for _ in range(2)]
          for _ in range(2)]
    for kh in range(3):
        a, p = kh // 2, kh % 2 if kh < 2 else 0
        if kh == 2:
            a, p = 1, 0
        for kw in range(3):
            b, q = (1, 0) if kw == 2 else (0, kw)
            blk = W[:, :, kh, kw].T
            ws[a][b] = jax.lax.dynamic_update_slice(
                ws[a][b], blk, ((p * 2 + q) * cin, 0))
    return jnp.stack([ws[0][0], ws[0][1], ws[1][0], ws[1][1]])


def _tail_body(h3_ref, wfc_ref, bfc_ref, g_ref, be_ref, buf_ref, o_ref):
    h = jax.lax.dot_general(h3_ref[...], wfc_ref[...], (((1,), (0,)), ((), ())),
                            preferred_element_type=jnp.float32) + bfc_ref[...]
    mu = jnp.mean(h, axis=1, keepdims=True)
    var = jnp.mean((h - mu) * (h - mu), axis=1, keepdims=True)
    hn = (h - mu) * jax.lax.rsqrt(var + 1e-5) * g_ref[...] + be_ref[...]
    reps = jnp.tanh(hn)
    buft = buf_ref[...]
    rr = jnp.sum(reps * reps, axis=1, keepdims=True)
    bb = jnp.sum(buft * buft, axis=0, keepdims=True)
    d2 = rr + bb - 2.0 * jax.lax.dot_general(
        reps, buft, (((1,), (0,)), ((), ())), preferred_element_type=jnp.float32)
    dist = jnp.sqrt(jnp.maximum(d2, 0.0))
    tot = jnp.zeros((B, 1), jnp.float32)
    for _ in range(K):
        m = jnp.min(dist, axis=1, keepdims=True)
        tot = tot + m
        i = jnp.argmin(dist, axis=1).astype(jnp.int32)
        hit = jax.lax.broadcasted_iota(jnp.int32, (B, BUFP), 1) == i[:, None]
        dist = jnp.where(hit, jnp.float32(jnp.inf), dist)
    o_ref[...] = -jnp.log(tot * (1.0 / K) + 1e-8)


def _ac_body(flat_ref, wh_ref, bh_ref, wa_ref, ba_ref, wv_ref, bv_ref,
             probs_ref, logp_ref, val_ref, acc_ref, *, nk):
    k = pl.program_id(0)

    @pl.when(k == 0)
    def _():
        acc_ref[...] = jnp.zeros_like(acc_ref)

    acc_ref[...] += jax.lax.dot_general(
        flat_ref[...], wh_ref[...], (((1,), (1,)), ((), ())),
        preferred_element_type=jnp.float32)

    @pl.when(k == nk - 1)
    def _():
        hid = jnp.maximum(acc_ref[...] + bh_ref[...], 0.0)
        logits = jax.lax.dot_general(hid, wa_ref[...], (((1,), (0,)), ((), ())),
                                     preferred_element_type=jnp.float32) + ba_ref[...]
        m = jnp.max(logits, axis=1, keepdims=True)
        e = jnp.exp(logits - m)
        s = jnp.sum(e, axis=1, keepdims=True)
        probs_ref[...] = e / s
        logp_ref[...] = logits - m - jnp.log(s)
        val_ref[...] = jax.lax.dot_general(
            hid, wv_ref[...], (((1,), (0,)), ((), ())),
            preferred_element_type=jnp.float32) + bv_ref[...]


def kernel(x, W1, b1, W2, b2, W3, b3, Wfc, bfc, gamma, beta, buffer,
           Wh, bh, Wa, ba, Wv, bv):
    flat = x.reshape(B, -1)
    kdim = flat.shape[1]
    nk = 8
    kc = kdim // nk
    wa_p = jnp.zeros((HID, NAP), jnp.float32).at[:, :NA].set(Wa.T)
    ba_p = jnp.full((1, NAP), -1e30, jnp.float32).at[:, :NA].set(ba)
    probs_p, logp_p, value = pl.pallas_call(
        functools.partial(_ac_body, nk=nk),
        grid=(nk,),
        in_specs=[
            pl.BlockSpec((B, kc), lambda k: (0, k)),
            pl.BlockSpec((HID, kc), lambda k: (0, k)),
            pl.BlockSpec((1, HID), lambda k: (0, 0)),
            pl.BlockSpec((HID, NAP), lambda k: (0, 0)),
            pl.BlockSpec((1, NAP), lambda k: (0, 0)),
            pl.BlockSpec((HID, 1), lambda k: (0, 0)),
            pl.BlockSpec((1, 1), lambda k: (0, 0)),
        ],
        out_specs=[
            pl.BlockSpec((B, NAP), lambda k: (0, 0)),
            pl.BlockSpec((B, NAP), lambda k: (0, 0)),
            pl.BlockSpec((B, 1), lambda k: (0, 0)),
        ],
        out_shape=[
            jax.ShapeDtypeStruct((B, NAP), jnp.float32),
            jax.ShapeDtypeStruct((B, NAP), jnp.float32),
            jax.ShapeDtypeStruct((B, 1), jnp.float32),
        ],
        scratch_shapes=[pltpu.VMEM((B, HID), jnp.float32)],
    )(flat, Wh, bh.reshape(1, HID), wa_p, ba_p, Wv.T, bv.reshape(1, 1))
    probs = probs_p[:, :NA]
    log_probs = logp_p[:, :NA]

    xt = x.transpose(0, 2, 3, 1)
    xp = jnp.pad(xt, ((0, 0), (1, 3), (1, 15), (0, 0)))
    xq = xp.reshape(B, G1, 4, G1C, 4, 4).transpose(0, 1, 3, 2, 4, 5)
    xqf = xq.reshape(B, G1, G1C, 64)

    w1q = _pack_w1(W1)
    w2q = _pack_w23(W2, 32, 64)
    w3q = _pack_w23(W3, 64, 128)
    b1q = jnp.tile(b1, 4).reshape(1, 128)

    h3 = pl.pallas_call(
        _enc_body,
        grid=(B // BT,),
        in_specs=[
            pl.BlockSpec((BT, G1, G1C, 64), lambda i: (i, 0, 0, 0)),
            pl.BlockSpec((4, 64, 128), lambda i: (0, 0, 0)),
            pl.BlockSpec((4, 128, 64), lambda i: (0, 0, 0)),
            pl.BlockSpec((4, 256, 128), lambda i: (0, 0, 0)),
            pl.BlockSpec((1, 128), lambda i: (0, 0)),
            pl.BlockSpec((1, 64), lambda i: (0, 0)),
            pl.BlockSpec((1, 128), lambda i: (0, 0)),
        ],
        out_specs=pl.BlockSpec((BT, 10, 10, 128), lambda i: (i, 0, 0, 0)),
        out_shape=jax.ShapeDtypeStruct((B, 10, 10, 128), jnp.float32),
    )(xqf, w1q, w2q, w3q, b1q, b2.reshape(1, 64), b3.reshape(1, 128))

    h3f = h3.reshape(B, -1)
    wfc_r = Wfc.reshape(LATENT, 128, 10, 10).transpose(2, 3, 1, 0).reshape(-1, LATENT)
    buf_p = jnp.full((BUFP, LATENT), 1e3, jnp.float32).at[:BUF].set(buffer)

    reward = pl.pallas_call(
        _tail_body,
        in_specs=[
            pl.BlockSpec((B, 12800), lambda: (0, 0)),
            pl.BlockSpec((12800, LATENT), lambda: (0, 0)),
            pl.BlockSpec((1, LATENT), lambda: (0, 0)),
            pl.BlockSpec((1, LATENT), lambda: (0, 0)),
            pl.BlockSpec((1, LATENT), lambda: (0, 0)),
            pl.BlockSpec((LATENT, BUFP), lambda: (0, 0)),
        ],
        out_specs=pl.BlockSpec((B, 1), lambda: (0, 0)),
        out_shape=jax.ShapeDtypeStruct((B, 1), jnp.float32),
    )(h3f, wfc_r, bfc.reshape(1, LATENT), gamma.reshape(1, LATENT),
      beta.reshape(1, LATENT), buf_p.T)

    return (probs, log_probs, value, reward.reshape(B))
